# baseline (speedup 1.0000x reference)
import functools

import jax
import jax.numpy as jnp
from jax import lax
from jax.experimental import pallas as pl
from jax.experimental.pallas import tpu as pltpu
from jax.experimental.pallas import tpu_sc as plsc

NUM_RELATIONS = 500
LANES = 16
NUM_CORES = 2
NUM_SUBCORES = 16
NUM_WORKERS = NUM_CORES * NUM_SUBCORES
WAVE = 128

_MESH = dict(core_axis_name="c", subcore_axis_name="s")


@functools.partial(jax.jit, static_argnames=("dim", "n_nodes"))
def _detile(ent_t, *, dim, n_nodes):
    tcols = (n_nodes + 127) // 128
    per_tile = (tcols + NUM_WORKERS - 1) // NUM_WORKERS + 2
    half = per_tile // 2
    out_rows = ((tcols * 128) // 2)
    row_w = 2 * dim
    mesh = plsc.VectorSubcoreMesh(**_MESH)

    @functools.partial(
        pl.kernel,
        out_type=jax.ShapeDtypeStruct((out_rows, row_w), jnp.float32),
        mesh=mesh,
        compiler_params=pltpu.CompilerParams(needs_layout_passes=False,
                                             disable_bounds_checks=True),
        scratch_types=[
            pltpu.VMEM((dim, 128), jnp.float32),
            pltpu.VMEM((dim, 128), jnp.float32),
            pltpu.VMEM((dim, row_w), jnp.float32),
            pltpu.VMEM((dim, row_w), jnp.float32),
            pltpu.SemaphoreType.DMA,
            pltpu.SemaphoreType.DMA,
            pltpu.SemaphoreType.DMA,
            pltpu.SemaphoreType.DMA,
        ],
    )
    def detiler(ent_hbm, out_hbm, slab_a, slab_b, blk_a, blk_b,
                sem_a, sem_b, sem_oa, sem_ob):
        wid = lax.axis_index("s") * NUM_CORES + lax.axis_index("c")
        span = 2 * half
        t_lo = jnp.minimum(wid * span, tcols - span)

        iota = lax.iota(jnp.int32, LANES)
        drows = [(k * LANES + iota) % dim for k in range(row_w // LANES)]
        zeros16 = jnp.zeros((LANES,), jnp.int32)

        def start_in(t, slab, sem):
            src = ent_hbm.at[:, pl.ds(pl.multiple_of(t * 128, 128), 128)]
            pltpu.async_copy(src, slab, sem)

        def drain(dst, sem):
            pltpu.make_async_copy(ent_hbm.at[:, pl.ds(0, 128)], dst,
                                  sem).wait()

        def transpose(slab, blk):
            def per_row(r, carry):
                r2 = r * 2
                for k in range(row_w // LANES):
                    lvec = zeros16 + (r2 + (k * LANES) // dim)
                    v = plsc.load_gather(slab, [drows[k], lvec])
                    blk[r, pl.ds(k * LANES, LANES)] = v
                return carry
            lax.fori_loop(0, dim, per_row, 0)

        start_in(t_lo, slab_a, sem_a)
        start_in(t_lo + half, slab_b, sem_b)

        def body(u, carry):
            for (slab, blk, sem_in, sem_out, off) in (
                    (slab_a, blk_a, sem_a, sem_oa, 0),
                    (slab_b, blk_b, sem_b, sem_ob, half)):
                t = t_lo + off + u
                drain(slab, sem_in)

                @pl.when(u > 0)
                def _():
                    drain(blk, sem_out)

                transpose(slab, blk)
                pltpu.async_copy(
                    blk, out_hbm.at[pl.ds(t * (row_w // 2), dim)], sem_out)

                @pl.when(u + 1 < half)
                def _():
                    start_in(t + 1, slab, sem_in)
            return carry

        lax.fori_loop(0, half, body, 0)
        drain(blk_a, sem_oa)
        drain(blk_b, sem_ob)

    return detiler(ent_t)


@functools.partial(jax.jit, static_argnames=("batch", "dim"))
def _score(s_idx, o_idx, t_idx, ent2, rel2, *, batch, dim):
    b_per_w = batch // NUM_WORKERS
    n_waves = b_per_w // WAVE
    row_w = 2 * dim
    mesh = plsc.VectorSubcoreMesh(**_MESH)

    @functools.partial(
        pl.kernel,
        out_type=jax.ShapeDtypeStruct((batch,), jnp.float32),
        mesh=mesh,
        compiler_params=pltpu.CompilerParams(needs_layout_passes=False),
        scratch_types=[
            pltpu.VMEM((b_per_w,), jnp.int32),
            pltpu.VMEM((b_per_w,), jnp.int32),
            pltpu.VMEM((b_per_w,), jnp.int32),
            pltpu.VMEM((WAVE * 2 * dim,), jnp.float32),
            pltpu.VMEM((WAVE * 2 * dim,), jnp.float32),
            pltpu.VMEM((WAVE * 2 * dim,), jnp.float32),
            pltpu.VMEM((LANES * b_per_w,), jnp.float32),
            pltpu.VMEM((b_per_w,), jnp.float32),
            pltpu.SemaphoreType.DMA,
        ],
    )
    def scorer(sidx_hbm, oidx_hbm, tidx_hbm, ent_hbm, rel_hbm, out_hbm,
               sidx_v, oidx_v, ridx_v, srows, orows, rrows, part_t, out_v,
               sem):
        wid = lax.axis_index("s") * NUM_CORES + lax.axis_index("c")
        base = wid * b_per_w

        pltpu.sync_copy(sidx_hbm.at[pl.ds(base, b_per_w)], sidx_v)
        pltpu.sync_copy(oidx_hbm.at[pl.ds(base, b_per_w)], oidx_v)
        pltpu.sync_copy(tidx_hbm.at[pl.ds(base, b_per_w)], ridx_v)

        for k in range(b_per_w // LANES):
            sl = pl.ds(k * LANES, LANES)
            ridx_v[sl] = lax.rem(ridx_v[sl],
                                 jnp.full((LANES,), NUM_RELATIONS, jnp.int32))

        lane_rows = lax.iota(jnp.int32, LANES) * b_per_w

        def fetch_group(g, w0):
            svec = sidx_v[pl.ds(w0 + g * LANES, LANES)]
            ovec = oidx_v[pl.ds(w0 + g * LANES, LANES)]
            rvec = ridx_v[pl.ds(w0 + g * LANES, LANES)]
            for l in range(LANES):
                i = g * LANES + l
                pltpu.async_copy(ent_hbm.at[svec[l] >> 1],
                                 srows.at[pl.ds(i * row_w, row_w)], sem)
                pltpu.async_copy(ent_hbm.at[ovec[l] >> 1],
                                 orows.at[pl.ds(i * row_w, row_w)], sem)
                pltpu.async_copy(rel_hbm.at[rvec[l] >> 1],
                                 rrows.at[pl.ds(i * row_w, row_w)], sem)
            return w0

        def drain_one(i, w0):
            for buf in (srows, orows, rrows):
                pltpu.make_async_copy(ent_hbm.at[0],
                                      buf.at[pl.ds(i * row_w, row_w)],
                                      sem).wait()
            return w0

        def compute_group(g, w0):
            svec = sidx_v[pl.ds(w0 + g * LANES, LANES)]
            ovec = oidx_v[pl.ds(w0 + g * LANES, LANES)]
            rvec = ridx_v[pl.ds(w0 + g * LANES, LANES)]
            for l in range(LANES):
                i = g * LANES + l
                soff = i * row_w + (svec[l] & 1) * dim
                ooff = i * row_w + (ovec[l] & 1) * dim
                roff = i * row_w + (rvec[l] & 1) * dim
                acc = jnp.zeros((LANES,), jnp.float32)
                for q in range(dim // LANES):
                    acc = acc + (srows[pl.ds(soff + q * LANES, LANES)] *
                                 rrows[pl.ds(roff + q * LANES, LANES)] *
                                 orows[pl.ds(ooff + q * LANES, LANES)])
                plsc.store_scatter(part_t, [lane_rows + w0 + i], acc)
            return w0

        for w in range(n_waves):
            lax.fori_loop(0, WAVE // LANES, fetch_group, w * WAVE)
            lax.fori_loop(0, WAVE, drain_one, w * WAVE)
            lax.fori_loop(0, WAVE // LANES, compute_group, w * WAVE)

        for g in range(b_per_w // LANES):
            sl = pl.ds(g * LANES, LANES)
            acc = part_t[pl.ds(g * LANES, LANES)]
            for j in range(1, LANES):
                acc = acc + part_t[pl.ds(j * b_per_w + g * LANES, LANES)]
            out_v[sl] = acc

        pltpu.sync_copy(out_v, out_hbm.at[pl.ds(base, b_per_w)])

    return scorer(s_idx, o_idx, t_idx, ent2, rel2)


def kernel(triples, entity_table, rel_table):
    s_idx = triples[:, 0].astype(jnp.int32)
    o_idx = triples[:, 1].astype(jnp.int32)
    t_idx = triples[:, 2].astype(jnp.int32)
    n_nodes, dim = entity_table.shape
    ent2 = _detile(entity_table.T, dim=dim, n_nodes=n_nodes)
    rel2 = rel_table.reshape(rel_table.shape[0] // 2, 2 * dim)
    return _score(s_idx, o_idx, t_idx, ent2, rel2,
                  batch=triples.shape[0], dim=dim)
